# baseline (speedup 1.0000x reference)
import functools

import jax
import jax.numpy as jnp
from jax import lax
from jax.experimental import pallas as pl
from jax.experimental.pallas import tpu as pltpu
from jax.experimental.pallas import tpu_sc as plsc

BATCH = 16384
DIM = 32
NSLOT = 8
PACK = 4
WIDE = PACK * DIM
NC, NS = 2, 16
NW = NC * NS
BPW = BATCH // NW
CHUNK = 64
NCHUNK = BPW // CHUNK
OUT_D = NSLOT * DIM
NBUF = 2
LANES = 16
GRP = CHUNK // LANES

_mesh = plsc.VectorSubcoreMesh(core_axis_name="c", subcore_axis_name="s")


@functools.partial(
    pl.kernel,
    out_type=jax.ShapeDtypeStruct((BATCH, OUT_D), jnp.float32),
    mesh=_mesh,
    scratch_types=[
        pltpu.VMEM((NSLOT * BPW,), jnp.int32),
        pltpu.VMEM((NSLOT * BPW,), jnp.int32),
        pltpu.VMEM((NSLOT, CHUNK, WIDE), jnp.float32),
        pltpu.VMEM((NBUF, CHUNK, OUT_D), jnp.float32),
        pltpu.SemaphoreType.DMA,
        pltpu.SemaphoreType.DMA,
        pltpu.SemaphoreType.DMA,
    ],
    compiler_params=pltpu.CompilerParams(needs_layout_passes=False),
)
def _gather_concat(q_hbm, rem_hbm, paper_hbm, pfield_hbm, author_hbm,
                   year_hbm, oa_hbm, out_hbm, q_v, rem_v, wide_v, asm_v,
                   gsem, wsem0, wsem1):
    wid = lax.axis_index("s") * NC + lax.axis_index("c")
    base = wid * BPW
    tables = (paper_hbm, pfield_hbm, pfield_hbm, author_hbm, author_hbm,
              author_hbm, year_hbm, oa_hbm)
    wsems = (wsem0, wsem1)
    pltpu.sync_copy(q_hbm.at[pl.ds(wid * NSLOT * BPW, NSLOT * BPW)], q_v)
    pltpu.sync_copy(rem_hbm.at[pl.ds(wid * NSLOT * BPW, NSLOT * BPW)], rem_v)

    lane = jnp.arange(LANES, dtype=jnp.int32)

    def issue_gathers(c):
        return [
            pltpu.async_copy(
                tab.at[q_v.at[pl.ds(s * BPW + c * CHUNK, CHUNK)]],
                wide_v.at[s], gsem)
            for s, tab in enumerate(tables)
        ]

    def repack(c, buf):
        def body(j, _):
            g = j // NSLOT
            s = j % NSLOT
            rows = g * LANES + lane
            rem16 = rem_v[pl.ds(s * BPW + c * CHUNK + g * LANES, LANES)]
            s_vec = jnp.full((LANES,), s, dtype=jnp.int32)
            buf_vec = jnp.full((LANES,), buf, dtype=jnp.int32)
            col0 = rem16 * DIM
            dst0 = s * DIM
            for k in range(DIM):
                val = plsc.load_gather(wide_v, [s_vec, rows, col0 + k])
                plsc.store_scatter(
                    asm_v, [buf_vec, rows,
                            jnp.full((LANES,), dst0 + k, dtype=jnp.int32)],
                    val)
            return 0
        lax.fori_loop(0, GRP * NSLOT, body, 0)

    writes = [None] * NBUF
    gathers = issue_gathers(0)
    for c in range(NCHUNK):
        buf = c % NBUF
        for g in gathers:
            g.wait()
        if writes[buf] is not None:
            writes[buf].wait()
        repack(c, buf)
        if c + 1 < NCHUNK:
            gathers = issue_gathers(c + 1)
        writes[buf] = pltpu.async_copy(
            asm_v.at[buf], out_hbm.at[pl.ds(base + c * CHUNK, CHUNK)],
            wsems[buf])
    for w in writes:
        if w is not None:
            w.wait()


def kernel(paperId, fieldsOfStudy_0, fieldsOfStudy_1, authors_0, authors_1,
           authors_2, year, isOpenAccess, paper_table, pfield_table,
           author_table, year_table, oa_table):
    idx = jnp.stack([paperId, fieldsOfStudy_0, fieldsOfStudy_1, authors_0,
                     authors_1, authors_2, year, isOpenAccess])
    idx = (idx.astype(jnp.int32)
              .reshape(NSLOT, NW, BPW)
              .transpose(1, 0, 2)
              .reshape(-1))
    q = idx // PACK
    rem = idx % PACK
    oa_pad = jnp.pad(oa_table, ((0, 1), (0, 0)))
    return _gather_concat(
        q, rem,
        paper_table.reshape(-1, WIDE), pfield_table.reshape(-1, WIDE),
        author_table.reshape(-1, WIDE), year_table.reshape(-1, WIDE),
        oa_pad.reshape(-1, WIDE))
